# baseline (speedup 1.0000x reference)
import functools

import jax
import jax.numpy as jnp
from jax import lax
from jax.experimental import pallas as pl
from jax.experimental.pallas import tpu as pltpu
from jax.experimental.pallas import tpu_sc as plsc

_B = 16384
_EMB = 64
_TAG = 32
_HID = 256
_OUT = 128

_NC = 2
_NS = 16
_NW = _NC * _NS
_BPW = _B // _NW
_TCH = 128

_BT = 1024
_F32 = jnp.float32


def _wid_base():
    wid = lax.axis_index("s") * _NC + lax.axis_index("c")
    return wid * _BPW


def _gather_tag(tab, idx_hbm, out_hbm, idx_v, tag_v, semt, base):
    sl = pl.ds(base, _BPW)
    pltpu.sync_copy(idx_hbm.at[sl], idx_v)
    for h in range(_BPW // _TCH):
        pltpu.async_copy(
            tab.at[idx_v.at[pl.ds(h * _TCH, _TCH)]],
            tag_v.at[pl.ds(h * _TCH, _TCH)], semt)
    pltpu.make_async_copy(tab.at[pl.ds(0, _BPW)], tag_v, semt).wait()
    pltpu.sync_copy(tag_v, out_hbm.at[sl])


def _sc_tags_body(cidx, clidx, gidx, ctab, cltab, gtab, ce_o, cle_o, ge_o,
                  idx_v, tag_v, semt):
    base = _wid_base()
    _gather_tag(ctab, cidx, ce_o, idx_v, tag_v, semt, base)
    _gather_tag(cltab, clidx, cle_o, idx_v, tag_v, semt, base)
    _gather_tag(gtab, gidx, ge_o, idx_v, tag_v, semt, base)


def _sc_big_body(iidx, itab, dep, ie_o, idx_v, big_v, sem):
    del dep
    base = _wid_base()
    _gather_tag(itab, iidx, ie_o, idx_v, big_v, sem, base)


@functools.cache
def _sc_tags():
    return pl.kernel(
        _sc_tags_body,
        mesh=plsc.VectorSubcoreMesh(core_axis_name="c", subcore_axis_name="s"),
        out_type=[
            jax.ShapeDtypeStruct((_B, 128), _F32),
            jax.ShapeDtypeStruct((_B, 128), _F32),
            jax.ShapeDtypeStruct((_B, 128), _F32),
        ],
        scratch_types=[
            pltpu.VMEM((_BPW,), jnp.int32),
            pltpu.VMEM((_BPW, 128), _F32),
            pltpu.SemaphoreType.DMA,
        ],
    )


@functools.cache
def _sc_big():
    return pl.kernel(
        _sc_big_body,
        mesh=plsc.VectorSubcoreMesh(core_axis_name="c", subcore_axis_name="s"),
        out_type=jax.ShapeDtypeStruct((_B, 128), _F32),
        scratch_types=[
            pltpu.VMEM((_BPW,), jnp.int32),
            pltpu.VMEM((_BPW, 128), _F32),
            pltpu.SemaphoreType.DMA,
        ],
    )


_K1 = _EMB + _EMB + 3 * _TAG
_H2 = 2 * _HID


def _tc_towers_body(ue2, ie2, uidx, iidx, ce, cle, ge, uprice, iprice,
                    W1c, uW1p, iW1p, b1c, W2c, b2c, out):
    uodd = (uidx[...] % 2)[:, None] == 1
    iodd = (iidx[...] % 2)[:, None] == 1
    ue = jnp.where(uodd, ue2[:, _EMB:], ue2[:, :_EMB])
    ie = jnp.where(iodd, ie2[:, _EMB:], ie2[:, :_EMB])
    x = jnp.concatenate(
        [ue, ie, ce[:, :_TAG], cle[:, :_TAG], ge[:, :_TAG]], axis=1)
    h = jnp.dot(x, W1c[...], preferred_element_type=_F32) + b1c[...]
    hu = h[:, :_HID] + uprice[...][:, None] * uW1p[...]
    hi = h[:, _HID:] + iprice[...][:, None] * iW1p[...]
    h = jnp.concatenate([jnp.maximum(hu, 0.0), jnp.maximum(hi, 0.0)], axis=1)
    y = jnp.dot(h, W2c[...], preferred_element_type=_F32) + b2c[...]
    uvec = y[:, :_OUT]
    ivec = y[:, _OUT:]
    un = jnp.sqrt(jnp.sum(uvec * uvec, axis=1))
    inrm = jnp.sqrt(jnp.sum(ivec * ivec, axis=1))
    denom = jnp.maximum(un, 1e-12) * jnp.maximum(inrm, 1e-12)
    out[...] = jnp.sum(uvec * ivec, axis=1) / denom


def _row_spec(cols):
    return pl.BlockSpec((_BT, cols), lambda i: (i, 0))


def _full_spec(r, c):
    return pl.BlockSpec((r, c), lambda i: (0, 0))


_tc_towers = pl.pallas_call(
    _tc_towers_body,
    grid=(_B // _BT,),
    in_specs=[
        _row_spec(128),
        _row_spec(128),
        pl.BlockSpec((_BT,), lambda i: (i,)),
        pl.BlockSpec((_BT,), lambda i: (i,)),
        _row_spec(128),
        _row_spec(128),
        _row_spec(128),
        pl.BlockSpec((_BT,), lambda i: (i,)),
        pl.BlockSpec((_BT,), lambda i: (i,)),
        _full_spec(_K1, _H2),
        _full_spec(1, _HID),
        _full_spec(1, _HID),
        _full_spec(1, _H2),
        _full_spec(_H2, 2 * _OUT),
        _full_spec(1, 2 * _OUT),
    ],
    out_specs=pl.BlockSpec((_BT,), lambda i: (i,)),
    out_shape=jax.ShapeDtypeStruct((_B,), _F32),
)


def kernel(user_idx, user_norm_price, item_idx, item_cat, item_color,
           item_graphic, item_norm_price, user_table, item_table, cat_table,
           color_table, graphic_table, uW1, ub1, uW2, ub2, iW1, ib1, iW2, ib2):
    i32 = jnp.int32
    pad = ((0, 0), (0, 128 - _TAG))
    ce, cle, ge = _sc_tags()(
        item_cat.astype(i32), item_color.astype(i32),
        item_graphic.astype(i32),
        jnp.pad(cat_table, pad), jnp.pad(color_table, pad),
        jnp.pad(graphic_table, pad))
    uidx = user_idx.astype(i32)
    iidx = item_idx.astype(i32)
    ue2 = _sc_big()(uidx // 2, user_table.reshape(-1, 128), ce)
    ie2 = _sc_big()(iidx // 2, item_table.reshape(-1, 128), cle)
    W1c = jnp.zeros((_K1, _H2), _F32)
    W1c = W1c.at[:_EMB, :_HID].set(uW1[:_EMB])
    W1c = W1c.at[_EMB:, _HID:].set(iW1[:_K1 - _EMB])
    W2c = jnp.zeros((_H2, 2 * _OUT), _F32)
    W2c = W2c.at[:_HID, :_OUT].set(uW2)
    W2c = W2c.at[_HID:, _OUT:].set(iW2)
    b1c = jnp.concatenate([ub1, ib1])[None, :]
    b2c = jnp.concatenate([ub2, ib2])[None, :]
    return _tc_towers(
        ue2, ie2, uidx, iidx, ce, cle, ge, user_norm_price, item_norm_price,
        W1c, uW1[_EMB:], iW1[_K1 - _EMB:], b1c, W2c, b2c)


# speedup vs baseline: 1.3323x; 1.3323x over previous
import functools

import jax
import jax.numpy as jnp
from jax import lax
from jax.experimental import pallas as pl
from jax.experimental.pallas import tpu as pltpu
from jax.experimental.pallas import tpu_sc as plsc

_B = 16384
_EMB = 64
_TAG = 32
_HID = 256
_OUT = 128

_NC = 2
_NS = 16
_NW = _NC * _NS
_BPW = _B // _NW
_TCH = 128

_BT = 2048
_F32 = jnp.float32


def _wid_base():
    wid = lax.axis_index("s") * _NC + lax.axis_index("c")
    return wid * _BPW


def _gather_big(tab, idx_hbm, out_hbm, idx_v, big_v, sem, base):
    rows = big_v.shape[0]
    pltpu.sync_copy(idx_hbm.at[pl.ds(base, _BPW)], idx_v)
    for r in range(_BPW // rows):

        def row16(j, _, r=r):
            v = idx_v[pl.ds(r * rows + j * 16, 16)]
            for k in range(16):
                pltpu.async_copy(tab.at[pl.ds(v[k], 1)],
                                 big_v.at[pl.ds(j * 16 + k, 1)], sem)
            return _

        lax.fori_loop(0, rows // 16, row16, 0)
        pltpu.make_async_copy(tab.at[pl.ds(0, rows)], big_v, sem).wait()
        pltpu.sync_copy(big_v, out_hbm.at[pl.ds(base + r * rows, rows)])


def _gather_tag(tab, idx_hbm, out_hbm, idx_v, tag_v, semt, base):
    sl = pl.ds(base, _BPW)
    pltpu.sync_copy(idx_hbm.at[sl], idx_v)
    for h in range(_BPW // _TCH):
        pltpu.async_copy(
            tab.at[idx_v.at[pl.ds(h * _TCH, _TCH)]],
            tag_v.at[pl.ds(h * _TCH, _TCH)], semt)
    pltpu.make_async_copy(tab.at[pl.ds(0, _BPW)], tag_v, semt).wait()
    pltpu.sync_copy(tag_v, out_hbm.at[sl])


def _sc_user_tags_body(uidx, cidx, clidx, gidx, utab, ctab, cltab, gtab,
                       ue_o, ce_o, cle_o, ge_o,
                       idx_v, big_v, tag_v, sem, semt):
    base = _wid_base()
    _gather_big(utab, uidx, ue_o, idx_v, big_v, sem, base)
    _gather_tag(ctab, cidx, ce_o, idx_v, tag_v, semt, base)
    _gather_tag(cltab, clidx, cle_o, idx_v, tag_v, semt, base)
    _gather_tag(gtab, gidx, ge_o, idx_v, tag_v, semt, base)


def _sc_item_body(iidx, itab, ie_o, idx_v, big_v, sem):
    base = _wid_base()
    _gather_big(itab, iidx, ie_o, idx_v, big_v, sem, base)


@functools.cache
def _sc_user_tags():
    return pl.kernel(
        _sc_user_tags_body,
        mesh=plsc.VectorSubcoreMesh(core_axis_name="c", subcore_axis_name="s"),
        out_type=[
            jax.ShapeDtypeStruct((_B, _EMB), _F32),
            jax.ShapeDtypeStruct((_B, 128), _F32),
            jax.ShapeDtypeStruct((_B, 128), _F32),
            jax.ShapeDtypeStruct((_B, 128), _F32),
        ],
        scratch_types=[
            pltpu.VMEM((_BPW,), jnp.int32),
            pltpu.VMEM((_BPW // 2, _EMB), _F32),
            pltpu.VMEM((_BPW, 128), _F32),
            pltpu.SemaphoreType.DMA,
            pltpu.SemaphoreType.DMA,
        ],
    )


@functools.cache
def _sc_item():
    return pl.kernel(
        _sc_item_body,
        mesh=plsc.VectorSubcoreMesh(core_axis_name="c", subcore_axis_name="s"),
        out_type=jax.ShapeDtypeStruct((_B, _EMB), _F32),
        scratch_types=[
            pltpu.VMEM((_BPW,), jnp.int32),
            pltpu.VMEM((_BPW, _EMB), _F32),
            pltpu.SemaphoreType.DMA,
        ],
    )


_K1 = _EMB + _EMB + 3 * _TAG
_H2 = 2 * _HID


def _tc_towers_body(ue, ie, ce, cle, ge, uprice, iprice,
                    W1c, uW1p, iW1p, b1c, W2c, b2c, out):
    x = jnp.concatenate(
        [ue[...], ie[...], ce[:, :_TAG], cle[:, :_TAG], ge[:, :_TAG]], axis=1)
    h = jnp.dot(x, W1c[...], preferred_element_type=_F32) + b1c[...]
    hu = h[:, :_HID] + uprice[...][:, None] * uW1p[...]
    hi = h[:, _HID:] + iprice[...][:, None] * iW1p[...]
    h = jnp.concatenate([jnp.maximum(hu, 0.0), jnp.maximum(hi, 0.0)], axis=1)
    y = jnp.dot(h, W2c[...], preferred_element_type=_F32) + b2c[...]
    uvec = y[:, :_OUT]
    ivec = y[:, _OUT:]
    un = jnp.sqrt(jnp.sum(uvec * uvec, axis=1))
    inrm = jnp.sqrt(jnp.sum(ivec * ivec, axis=1))
    denom = jnp.maximum(un, 1e-12) * jnp.maximum(inrm, 1e-12)
    out[...] = jnp.sum(uvec * ivec, axis=1) / denom


def _row_spec(cols):
    return pl.BlockSpec((_BT, cols), lambda i: (i, 0))


def _full_spec(r, c):
    return pl.BlockSpec((r, c), lambda i: (0, 0))


_tc_towers = pl.pallas_call(
    _tc_towers_body,
    grid=(_B // _BT,),
    in_specs=[
        _row_spec(_EMB),
        _row_spec(_EMB),
        _row_spec(128),
        _row_spec(128),
        _row_spec(128),
        pl.BlockSpec((_BT,), lambda i: (i,)),
        pl.BlockSpec((_BT,), lambda i: (i,)),
        _full_spec(_K1, _H2),
        _full_spec(1, _HID),
        _full_spec(1, _HID),
        _full_spec(1, _H2),
        _full_spec(_H2, 2 * _OUT),
        _full_spec(1, 2 * _OUT),
    ],
    out_specs=pl.BlockSpec((_BT,), lambda i: (i,)),
    out_shape=jax.ShapeDtypeStruct((_B,), _F32),
)


def kernel(user_idx, user_norm_price, item_idx, item_cat, item_color,
           item_graphic, item_norm_price, user_table, item_table, cat_table,
           color_table, graphic_table, uW1, ub1, uW2, ub2, iW1, ib1, iW2, ib2):
    i32 = jnp.int32
    pad = ((0, 0), (0, 128 - _TAG))
    ie = _sc_item()(item_idx.astype(i32), item_table)
    ue, ce, cle, ge = _sc_user_tags()(
        user_idx.astype(i32), item_cat.astype(i32),
        item_color.astype(i32), item_graphic.astype(i32),
        user_table,
        jnp.pad(cat_table, pad), jnp.pad(color_table, pad),
        jnp.pad(graphic_table, pad))
    W1c = jnp.zeros((_K1, _H2), _F32)
    W1c = W1c.at[:_EMB, :_HID].set(uW1[:_EMB])
    W1c = W1c.at[_EMB:, _HID:].set(iW1[:_K1 - _EMB])
    W2c = jnp.zeros((_H2, 2 * _OUT), _F32)
    W2c = W2c.at[:_HID, :_OUT].set(uW2)
    W2c = W2c.at[_HID:, _OUT:].set(iW2)
    b1c = jnp.concatenate([ub1, ib1])[None, :]
    b2c = jnp.concatenate([ub2, ib2])[None, :]
    return _tc_towers(
        ue, ie, ce, cle, ge, user_norm_price, item_norm_price,
        W1c, uW1[_EMB:], iW1[_K1 - _EMB:], b1c, W2c, b2c)
